# Initial kernel scaffold; baseline (speedup 1.0000x reference)
import jax
import jax.numpy as jnp
from jax.experimental import pallas as pl


def kernel(x, edge_index, W1, b1, W2, b2, W3, b3):
    raise NotImplementedError("write your pallas kernel here")


# speedup vs baseline: 12.8837x; 12.8837x over previous
import functools

import jax
import jax.numpy as jnp
from jax import lax
from jax.experimental import pallas as pl
from jax.experimental.pallas import tpu as pltpu
from jax.experimental.pallas import tpu_sc as plsc

_N = 10000
_NP = 10240
_E = 320000
_D = 128
_H = 128
_C = 16

_CHUNK = 128
_NCHUNKS = _E // _CHUNK
_NWORKERS = 32
_TILES = 16
_RPT = _NP // _TILES
_BASE_CHUNKS = _NCHUNKS // _NWORKERS
_REM_CHUNKS = _NCHUNKS % _NWORKERS

_BLK = 1024
_GRID = _NP // _BLK


def _make_edge_agg(width):
    mesh = plsc.VectorSubcoreMesh(core_axis_name="c", subcore_axis_name="s")

    @functools.partial(
        pl.kernel,
        mesh=mesh,
        out_type=jax.ShapeDtypeStruct((2 * _NP, width), jnp.float32),
        scratch_types=[
            pltpu.VMEM_SHARED((_NP, width), jnp.float32),
            pltpu.VMEM((1, _CHUNK), jnp.int32),
            pltpu.VMEM((1, _CHUNK), jnp.int32),
            pltpu.VMEM((_CHUNK, width), jnp.float32),
            pltpu.SemaphoreType.DMA,
        ],
    )
    def agg(y_hbm, src_hbm, dst_hbm, zeros_hbm, out_hbm,
            acc, src_v, dst_v, rows_v, sem):
        c = lax.axis_index("c")
        s = lax.axis_index("s")
        wid = s * 2 + c
        r0 = s * _RPT
        pltpu.sync_copy(zeros_hbm.at[pl.ds(r0, _RPT)], acc.at[pl.ds(r0, _RPT)])
        plsc.subcore_barrier()

        nch = _BASE_CHUNKS + jnp.where(wid < _REM_CHUNKS, 1, 0)

        def body(i, carry):
            off = (i * _NWORKERS + wid) * _CHUNK
            pltpu.sync_copy(src_hbm.at[pl.ds(off, _CHUNK)], src_v.at[0])
            pltpu.async_copy(y_hbm.at[src_v.at[0]], rows_v, sem).wait()
            pltpu.sync_copy(dst_hbm.at[pl.ds(off, _CHUNK)], dst_v.at[0])
            pltpu.sync_copy(rows_v, acc.at[dst_v.at[0]], add=True)
            return carry

        lax.fori_loop(0, nch, body, 0)
        plsc.subcore_barrier()
        pltpu.sync_copy(acc.at[pl.ds(r0, _RPT)],
                        out_hbm.at[pl.ds(c * _NP + r0, _RPT)])

    return agg


_edge_agg_h = _make_edge_agg(_H)


_DEGW = 128

@functools.partial(
    pl.kernel,
    mesh=plsc.VectorSubcoreMesh(core_axis_name="c", subcore_axis_name="s"),
    out_type=jax.ShapeDtypeStruct((2 * _NP, _DEGW), jnp.float32),
    scratch_types=[
        pltpu.VMEM_SHARED((_NP, _DEGW), jnp.float32),
        pltpu.VMEM((1, _CHUNK), jnp.int32),
        pltpu.VMEM((_CHUNK, _DEGW), jnp.float32),
        pltpu.SemaphoreType.DMA,
    ],
)
def _deg_count(dst_hbm, zeros_hbm, ones_hbm, out_hbm, acc, dst_v, ones_v, sem):
    c = lax.axis_index("c")
    s = lax.axis_index("s")
    wid = s * 2 + c
    r0 = s * _RPT
    pltpu.sync_copy(zeros_hbm.at[pl.ds(r0, _RPT)], acc.at[pl.ds(r0, _RPT)])
    pltpu.sync_copy(ones_hbm, ones_v)
    plsc.subcore_barrier()

    nch = _BASE_CHUNKS + jnp.where(wid < _REM_CHUNKS, 1, 0)

    def body(i, carry):
        off = (i * _NWORKERS + wid) * _CHUNK
        pltpu.sync_copy(dst_hbm.at[pl.ds(off, _CHUNK)], dst_v.at[0])
        pltpu.sync_copy(ones_v, acc.at[dst_v.at[0]], add=True)
        return carry

    lax.fori_loop(0, nch, body, 0)
    plsc.subcore_barrier()
    pltpu.sync_copy(acc.at[pl.ds(r0, _RPT)],
                    out_hbm.at[pl.ds(c * _NP + r0, _RPT)])


def _dinv_block(d0_ref, d1_ref):
    tot = d0_ref[:, 0:1] + d1_ref[:, 0:1] + 1.0
    return lax.rsqrt(tot)


def _k1_body(d0_ref, d1_ref, x_ref, w_ref, o_ref):
    dinv = _dinv_block(d0_ref, d1_ref)
    o_ref[...] = jnp.dot(x_ref[...], w_ref[...],
                         preferred_element_type=jnp.float32) * dinv


def _k_mid_body(d0_ref, d1_ref, p0_ref, p1_ref, y_ref, b_ref, w_ref, o_ref):
    dinv = _dinv_block(d0_ref, d1_ref)
    h = jax.nn.relu(dinv * (p0_ref[...] + p1_ref[...] + y_ref[...]) + b_ref[...])
    o_ref[...] = jnp.dot(h, w_ref[...],
                         preferred_element_type=jnp.float32) * dinv


def _k_pre_body(d0_ref, d1_ref, p0_ref, p1_ref, y_ref, b_ref, o_ref):
    dinv = _dinv_block(d0_ref, d1_ref)
    h = jax.nn.relu(dinv * (p0_ref[...] + p1_ref[...] + y_ref[...]) + b_ref[...])
    o_ref[...] = h * dinv


def _k_final_body(d0_ref, d1_ref, p0_ref, p1_ref, z_ref, w_ref, b_ref, o_ref):
    dinv = _dinv_block(d0_ref, d1_ref)
    agg = p0_ref[...] + p1_ref[...] + z_ref[...]
    o_ref[...] = dinv * jnp.dot(agg, w_ref[...],
                                preferred_element_type=jnp.float32) + b_ref[...]


def _deg_specs():
    return [
        pl.BlockSpec((_BLK, _DEGW), lambda i: (i, 0)),
        pl.BlockSpec((_BLK, _DEGW), lambda i: (i + _GRID, 0)),
    ]


def _part_specs(width):
    return [
        pl.BlockSpec((_BLK, width), lambda i: (i, 0)),
        pl.BlockSpec((_BLK, width), lambda i: (i + _GRID, 0)),
    ]


def _tc_k1(degp, x, w):
    return pl.pallas_call(
        _k1_body,
        grid=(_GRID,),
        in_specs=_deg_specs() + [
            pl.BlockSpec((_BLK, _D), lambda i: (i, 0)),
            pl.BlockSpec((_D, _H), lambda i: (0, 0)),
        ],
        out_specs=pl.BlockSpec((_BLK, _H), lambda i: (i, 0)),
        out_shape=jax.ShapeDtypeStruct((_NP, _H), jnp.float32),
    )(degp, degp, x, w)


def _tc_k_mid(degp, part, y, b, w, wout):
    return pl.pallas_call(
        _k_mid_body,
        grid=(_GRID,),
        in_specs=_deg_specs() + _part_specs(_H) + [
            pl.BlockSpec((_BLK, _H), lambda i: (i, 0)),
            pl.BlockSpec((1, _H), lambda i: (0, 0)),
            pl.BlockSpec((_H, wout), lambda i: (0, 0)),
        ],
        out_specs=pl.BlockSpec((_BLK, wout), lambda i: (i, 0)),
        out_shape=jax.ShapeDtypeStruct((_NP, wout), jnp.float32),
    )(degp, degp, part, part, y, b, w)


def _tc_k_pre(degp, part, y, b):
    return pl.pallas_call(
        _k_pre_body,
        grid=(_GRID,),
        in_specs=_deg_specs() + _part_specs(_H) + [
            pl.BlockSpec((_BLK, _H), lambda i: (i, 0)),
            pl.BlockSpec((1, _H), lambda i: (0, 0)),
        ],
        out_specs=pl.BlockSpec((_BLK, _H), lambda i: (i, 0)),
        out_shape=jax.ShapeDtypeStruct((_NP, _H), jnp.float32),
    )(degp, degp, part, part, y, b)


def _tc_k_final(degp, part, z, w, b):
    return pl.pallas_call(
        _k_final_body,
        grid=(_GRID,),
        in_specs=_deg_specs() + _part_specs(_H) + [
            pl.BlockSpec((_BLK, _H), lambda i: (i, 0)),
            pl.BlockSpec((_H, _C), lambda i: (0, 0)),
            pl.BlockSpec((1, _C), lambda i: (0, 0)),
        ],
        out_specs=pl.BlockSpec((_BLK, _C), lambda i: (i, 0)),
        out_shape=jax.ShapeDtypeStruct((_NP, _C), jnp.float32),
    )(degp, degp, part, part, z, w, b)


def kernel(x, edge_index, W1, b1, W2, b2, W3, b3):
    src = edge_index[0]
    dst = edge_index[1]

    x_pad = jnp.concatenate([x, jnp.zeros((_NP - _N, _D), jnp.float32)], axis=0)
    zeros_h = jnp.zeros((_NP, _H), jnp.float32)
    zeros_d = jnp.zeros((_NP, _DEGW), jnp.float32)
    ones_d = jnp.ones((_CHUNK, _DEGW), jnp.float32)

    degp = _deg_count(dst, zeros_d, ones_d)

    y1 = _tc_k1(degp, x_pad, W1)
    p1 = _edge_agg_h(y1, src, dst, zeros_h)
    y2 = _tc_k_mid(degp, p1, y1, b1.reshape(1, _H), W2, _H)
    p2 = _edge_agg_h(y2, src, dst, zeros_h)
    z = _tc_k_pre(degp, p2, y2, b2.reshape(1, _H))
    p3 = _edge_agg_h(z, src, dst, zeros_h)
    out = _tc_k_final(degp, p3, z, W3, b3.reshape(1, _C))
    return out[:_N]
